# Initial kernel scaffold; baseline (speedup 1.0000x reference)
import jax
import jax.numpy as jnp
from jax.experimental import pallas as pl


def kernel(x, edge_index, W_in, b_in, W1, b1, W2, b2):
    raise NotImplementedError("write your pallas kernel here")


# speedup vs baseline: 12.8257x; 12.8257x over previous
import functools

import jax
import jax.numpy as jnp
from jax import lax
from jax.experimental import pallas as pl
from jax.experimental.pallas import tpu as pltpu
from jax.experimental.pallas import tpu_sc as plsc

N_NODES = 10000
DIM = 128
NUM_EDGES = 320000
NC = 2
NS = 16
LANES = 16
NTILE = NC * NS

EPT = NUM_EDGES // NTILE
K = 80
CPT = EPT // K

RSTEP = 624
RSPAN = 640

_mesh = plsc.VectorSubcoreMesh(core_axis_name="c", subcore_axis_name="s")


def _zero_fill(zbuf):
    zero16 = jnp.zeros((LANES,), dtype=jnp.float32)

    @pl.loop(0, 128)
    def _(r):
        @pl.loop(0, DIM, step=LANES)
        def _(c0):
            zbuf[r, pl.ds(c0, LANES)] = zero16


def _zero_shared(zbuf, sh, sid):
    @pl.loop(0, RSPAN, step=128)
    def _(r):
        pltpu.sync_copy(zbuf, sh.at[pl.ds(sid * RSTEP + r, 128)])


def _copy_out(sh, out_hbm, cid, sid):
    pltpu.sync_copy(
        sh.at[pl.ds(sid * RSTEP, RSPAN)],
        out_hbm.at[cid, pl.ds(sid * RSTEP, RSPAN)],
    )


@functools.partial(
    pl.kernel,
    out_type=jax.ShapeDtypeStruct((NC, N_NODES, DIM), jnp.float32),
    mesh=_mesh,
    scratch_types=[
        pltpu.VMEM((K,), jnp.int32),
        pltpu.VMEM((K, DIM), jnp.float32),
        pltpu.VMEM((128, DIM), jnp.float32),
        pltpu.VMEM_SHARED((N_NODES, DIM), jnp.float32),
    ],
)
def _deg_sc(dst_hbm, out_hbm, didx, ones_v, zbuf, deg_sh):
    cid = lax.axis_index("c")
    sid = lax.axis_index("s")
    wid = cid * NS + sid
    one16 = jnp.full((LANES,), 1.0, dtype=jnp.float32)

    @pl.loop(0, K)
    def _(r):
        @pl.loop(0, DIM, step=LANES)
        def _(c0):
            ones_v[r, pl.ds(c0, LANES)] = one16

    _zero_fill(zbuf)
    _zero_shared(zbuf, deg_sh, sid)
    plsc.subcore_barrier()

    ebase = wid * EPT

    @pl.loop(0, CPT)
    def _(j):
        pltpu.sync_copy(dst_hbm.at[pl.ds(ebase + j * K, K)], didx)
        pltpu.sync_copy(ones_v, deg_sh.at[didx], add=True)

    plsc.subcore_barrier()
    _copy_out(deg_sh, out_hbm, cid, sid)


@functools.partial(
    pl.kernel,
    out_type=jax.ShapeDtypeStruct((NC, N_NODES, DIM), jnp.float32),
    mesh=_mesh,
    scratch_types=[
        pltpu.VMEM((K,), jnp.int32),
        pltpu.VMEM((K,), jnp.int32),
        pltpu.VMEM((K, DIM), jnp.float32),
        pltpu.VMEM((128, DIM), jnp.float32),
        pltpu.VMEM_SHARED((N_NODES, DIM), jnp.float32),
        pltpu.SemaphoreType.DMA,
    ],
)
def _agg_sc(u_hbm, src_hbm, dst_hbm, out_hbm, sidx, didx, rows, zbuf, agg_sh, sem):
    cid = lax.axis_index("c")
    sid = lax.axis_index("s")
    wid = cid * NS + sid

    _zero_fill(zbuf)
    _zero_shared(zbuf, agg_sh, sid)
    plsc.subcore_barrier()

    ebase = wid * EPT

    @pl.loop(0, CPT)
    def _(j):
        pltpu.sync_copy(src_hbm.at[pl.ds(ebase + j * K, K)], sidx)
        pltpu.sync_copy(dst_hbm.at[pl.ds(ebase + j * K, K)], didx)
        pltpu.async_copy(u_hbm.at[sidx], rows, sem).wait()
        pltpu.sync_copy(rows, agg_sh.at[didx], add=True)

    plsc.subcore_barrier()
    _copy_out(agg_sh, out_hbm, cid, sid)


def _dinv_from(degp_ref):
    deg = degp_ref[0, :, 0:1] + degp_ref[1, :, 0:1] + 1.0
    return lax.rsqrt(deg)


def _tc_in_proj_body(x_ref, w_ref, b_ref, o_ref):
    o_ref[...] = (
        jnp.dot(x_ref[...], w_ref[...], preferred_element_type=jnp.float32)
        + b_ref[...]
    )


def _tc_scale_mm_body(h_ref, w_ref, degp_ref, o_ref):
    dinv = _dinv_from(degp_ref)
    o_ref[...] = (
        jnp.dot(h_ref[...], w_ref[...], preferred_element_type=jnp.float32) * dinv
    )


def _tc_layer_body(aggp_ref, u_ref, degp_ref, b_ref, w_ref, o_ref):
    dinv = _dinv_from(degp_ref)
    agg = aggp_ref[0] + aggp_ref[1] + u_ref[...]
    hl = jnp.maximum(agg * dinv + b_ref[...], 0.0)
    o_ref[...] = (
        jnp.dot(hl, w_ref[...], preferred_element_type=jnp.float32) * dinv
    )


def _tc_final_body(aggp_ref, u_ref, degp_ref, b_ref, o_ref):
    dinv = _dinv_from(degp_ref)
    agg = aggp_ref[0] + aggp_ref[1] + u_ref[...]
    o_ref[...] = jnp.maximum(agg * dinv + b_ref[...], 0.0)


_out_nd = jax.ShapeDtypeStruct((N_NODES, DIM), jnp.float32)

_tc_in_proj = pl.pallas_call(_tc_in_proj_body, out_shape=_out_nd)
_tc_scale_mm = pl.pallas_call(_tc_scale_mm_body, out_shape=_out_nd)
_tc_layer = pl.pallas_call(_tc_layer_body, out_shape=_out_nd)
_tc_final = pl.pallas_call(_tc_final_body, out_shape=_out_nd)


def kernel(x, edge_index, W_in, b_in, W1, b1, W2, b2):
    src = edge_index[0]
    dst = edge_index[1]
    b_in2 = b_in.reshape(1, DIM)
    b1_2 = b1.reshape(1, DIM)
    b2_2 = b2.reshape(1, DIM)

    degp = _deg_sc(dst)
    h = _tc_in_proj(x, W_in, b_in2)
    u1 = _tc_scale_mm(h, W1, degp)
    aggp1 = _agg_sc(u1, src, dst)
    u2 = _tc_layer(aggp1, u1, degp, b1_2, W2)
    aggp2 = _agg_sc(u2, src, dst)
    return _tc_final(aggp2, u2, degp, b2_2)


# speedup vs baseline: 27.9541x; 2.1795x over previous
import functools

import jax
import jax.numpy as jnp
from jax import lax
from jax.experimental import pallas as pl
from jax.experimental.pallas import tpu as pltpu
from jax.experimental.pallas import tpu_sc as plsc

N_NODES = 10000
DIM = 128
NUM_EDGES = 320000
NC = 2
NS = 16
LANES = 16
NTILE = NC * NS

EPT = NUM_EDGES // NTILE
K = 80
CPT = EPT // K

RSTEP = 624
RSPAN = 640

_mesh = plsc.VectorSubcoreMesh(core_axis_name="c", subcore_axis_name="s")


def _zero_fill(zbuf):
    zero16 = jnp.zeros((LANES,), dtype=jnp.float32)

    @pl.loop(0, 64)
    def _(r):
        @pl.loop(0, DIM, step=LANES)
        def _(c0):
            zbuf[r, pl.ds(c0, LANES)] = zero16


def _zero_shared(zbuf, sh, sid):
    @pl.loop(0, RSPAN, step=64)
    def _(r):
        pltpu.sync_copy(zbuf, sh.at[pl.ds(sid * RSTEP + r, 64)])


def _copy_out(sh, out_hbm, cid, sid):
    pltpu.sync_copy(
        sh.at[pl.ds(sid * RSTEP, RSPAN)],
        out_hbm.at[cid, pl.ds(sid * RSTEP, RSPAN)],
    )


@functools.partial(
    pl.kernel,
    out_type=jax.ShapeDtypeStruct((NC, N_NODES, DIM), jnp.float32),
    mesh=_mesh,
    scratch_types=[
        pltpu.VMEM((CPT, K), jnp.int32),
        pltpu.VMEM((K, DIM), jnp.float32),
        pltpu.VMEM((64, DIM), jnp.float32),
        pltpu.VMEM_SHARED((N_NODES, DIM), jnp.float32),
    ],
)
def _deg_sc(dst3_hbm, out_hbm, didx2, ones_v, zbuf, deg_sh):
    cid = lax.axis_index("c")
    sid = lax.axis_index("s")
    wid = cid * NS + sid
    one16 = jnp.full((LANES,), 1.0, dtype=jnp.float32)

    @pl.loop(0, K)
    def _(r):
        @pl.loop(0, DIM, step=LANES)
        def _(c0):
            ones_v[r, pl.ds(c0, LANES)] = one16

    pltpu.sync_copy(dst3_hbm.at[wid], didx2)
    _zero_fill(zbuf)
    _zero_shared(zbuf, deg_sh, sid)
    plsc.subcore_barrier()

    @pl.loop(0, CPT)
    def _(j):
        pltpu.sync_copy(ones_v, deg_sh.at[didx2.at[j]], add=True)

    plsc.subcore_barrier()
    _copy_out(deg_sh, out_hbm, cid, sid)


@functools.partial(
    pl.kernel,
    out_type=jax.ShapeDtypeStruct((NC, N_NODES, DIM), jnp.float32),
    mesh=_mesh,
    scratch_types=[
        pltpu.VMEM((CPT, K), jnp.int32),
        pltpu.VMEM((1, K), jnp.int32),
        pltpu.VMEM((1, K), jnp.int32),
        pltpu.VMEM((K, DIM), jnp.float32),
        pltpu.VMEM((K, DIM), jnp.float32),
        pltpu.VMEM_SHARED((N_NODES, DIM), jnp.float32),
        pltpu.SemaphoreType.DMA,
        pltpu.SemaphoreType.DMA,
        pltpu.SemaphoreType.DMA,
        pltpu.SemaphoreType.DMA,
    ],
)
def _agg_sc(u_hbm, src3_hbm, dst3_hbm, out_hbm,
            sidx2, didxA, didxB, rows0, rows1, agg_sh,
            sem0, sem1, semA, semB):
    cid = lax.axis_index("c")
    sid = lax.axis_index("s")
    wid = cid * NS + sid

    pltpu.sync_copy(src3_hbm.at[wid], sidx2)
    zero16 = jnp.zeros((LANES,), dtype=jnp.float32)

    @pl.loop(0, K)
    def _(r):
        @pl.loop(0, DIM, step=LANES)
        def _(c0):
            rows0[r, pl.ds(c0, LANES)] = zero16

    @pl.loop(0, RSPAN, step=K)
    def _(r):
        pltpu.sync_copy(rows0, agg_sh.at[pl.ds(sid * RSTEP + r, K)])

    plsc.subcore_barrier()

    def _dload(j, dbuf, dsem):
        pltpu.async_copy(dst3_hbm.at[wid, pl.ds(j, 1)], dbuf, dsem)

    def _dwait(j, dbuf, dsem):
        pltpu.make_async_copy(dst3_hbm.at[wid, pl.ds(j, 1)], dbuf, dsem).wait()

    def _gather(j, buf, sem):
        pltpu.async_copy(u_hbm.at[sidx2.at[j]], buf, sem)

    def _gwait(j, buf, sem):
        pltpu.make_async_copy(u_hbm.at[sidx2.at[j]], buf, sem).wait()

    def _scatter(j, buf, dbuf, dsem):
        _dwait(j, dbuf, dsem)
        pltpu.sync_copy(buf, agg_sh.at[dbuf.at[0]], add=True)

    _dload(0, didxA, semA)
    _gather(0, rows0, sem0)

    @pl.loop(0, CPT - 3, step=2)
    def _(j):
        _dload(j + 1, didxB, semB)
        _gather(j + 1, rows1, sem1)
        _gwait(j, rows0, sem0)
        _scatter(j, rows0, didxA, semA)
        _dload(j + 2, didxA, semA)
        _gather(j + 2, rows0, sem0)
        _gwait(j + 1, rows1, sem1)
        _scatter(j + 1, rows1, didxB, semB)

    _dload(CPT - 2, didxB, semB)
    _gather(CPT - 2, rows1, sem1)
    _gwait(CPT - 3, rows0, sem0)
    _scatter(CPT - 3, rows0, didxA, semA)
    _dload(CPT - 1, didxA, semA)
    _gather(CPT - 1, rows0, sem0)
    _gwait(CPT - 2, rows1, sem1)
    _scatter(CPT - 2, rows1, didxB, semB)
    _gwait(CPT - 1, rows0, sem0)
    _scatter(CPT - 1, rows0, didxA, semA)

    plsc.subcore_barrier()
    _copy_out(agg_sh, out_hbm, cid, sid)


def _dinv_from(degp_ref):
    deg = degp_ref[0, :, 0:1] + degp_ref[1, :, 0:1] + 1.0
    return lax.rsqrt(deg)


def _tc_in_proj_body(x_ref, w_ref, b_ref, o_ref):
    o_ref[...] = (
        jnp.dot(x_ref[...], w_ref[...], preferred_element_type=jnp.float32)
        + b_ref[...]
    )


def _tc_scale_mm_body(h_ref, w_ref, degp_ref, o_ref):
    dinv = _dinv_from(degp_ref)
    o_ref[...] = (
        jnp.dot(h_ref[...], w_ref[...], preferred_element_type=jnp.float32) * dinv
    )


def _tc_layer_body(aggp_ref, u_ref, degp_ref, b_ref, w_ref, o_ref):
    dinv = _dinv_from(degp_ref)
    agg = aggp_ref[0] + aggp_ref[1] + u_ref[...]
    hl = jnp.maximum(agg * dinv + b_ref[...], 0.0)
    o_ref[...] = (
        jnp.dot(hl, w_ref[...], preferred_element_type=jnp.float32) * dinv
    )


def _tc_final_body(aggp_ref, u_ref, degp_ref, b_ref, o_ref):
    dinv = _dinv_from(degp_ref)
    agg = aggp_ref[0] + aggp_ref[1] + u_ref[...]
    o_ref[...] = jnp.maximum(agg * dinv + b_ref[...], 0.0)


_out_nd = jax.ShapeDtypeStruct((N_NODES, DIM), jnp.float32)

_tc_in_proj = pl.pallas_call(_tc_in_proj_body, out_shape=_out_nd)
_tc_scale_mm = pl.pallas_call(_tc_scale_mm_body, out_shape=_out_nd)
_tc_layer = pl.pallas_call(_tc_layer_body, out_shape=_out_nd)
_tc_final = pl.pallas_call(_tc_final_body, out_shape=_out_nd)


def kernel(x, edge_index, W_in, b_in, W1, b1, W2, b2):
    src3 = edge_index[0].reshape(NTILE, CPT, K)
    dst3 = edge_index[1].reshape(NTILE, CPT, K)
    b_in2 = b_in.reshape(1, DIM)
    b1_2 = b1.reshape(1, DIM)
    b2_2 = b2.reshape(1, DIM)

    degp = _deg_sc(dst3)
    h = _tc_in_proj(x, W_in, b_in2)
    u1 = _tc_scale_mm(h, W1, degp)
    aggp1 = _agg_sc(u1, src3, dst3)
    u2 = _tc_layer(aggp1, u1, degp, b1_2, W2)
    aggp2 = _agg_sc(u2, src3, dst3)
    return _tc_final(aggp2, u2, degp, b2_2)


# speedup vs baseline: 28.0270x; 1.0026x over previous
import functools

import jax
import jax.numpy as jnp
from jax import lax
from jax.experimental import pallas as pl
from jax.experimental.pallas import tpu as pltpu
from jax.experimental.pallas import tpu_sc as plsc

N_NODES = 10000
DIM = 128
NUM_EDGES = 320000
NC = 2
NS = 16
LANES = 16
NTILE = NC * NS

EPT = NUM_EDGES // NTILE
K = 80
CPT = EPT // K

RSTEP = 624
RSPAN = 640

_mesh = plsc.VectorSubcoreMesh(core_axis_name="c", subcore_axis_name="s")


def _zero_fill(zbuf):
    zero16 = jnp.zeros((LANES,), dtype=jnp.float32)

    @pl.loop(0, 64)
    def _(r):
        @pl.loop(0, DIM, step=LANES)
        def _(c0):
            zbuf[r, pl.ds(c0, LANES)] = zero16


def _zero_shared(zbuf, sh, sid):
    @pl.loop(0, RSPAN, step=64)
    def _(r):
        pltpu.sync_copy(zbuf, sh.at[pl.ds(sid * RSTEP + r, 64)])


def _copy_out(sh, out_hbm, cid, sid):
    pltpu.sync_copy(
        sh.at[pl.ds(sid * RSTEP, RSPAN)],
        out_hbm.at[cid, pl.ds(sid * RSTEP, RSPAN)],
    )


@functools.partial(
    pl.kernel,
    out_type=jax.ShapeDtypeStruct((NC, N_NODES, DIM), jnp.float32),
    mesh=_mesh,
    scratch_types=[
        pltpu.VMEM((CPT, K), jnp.int32),
        pltpu.VMEM((K, DIM), jnp.float32),
        pltpu.VMEM((64, DIM), jnp.float32),
        pltpu.VMEM_SHARED((N_NODES, DIM), jnp.float32),
        pltpu.SemaphoreType.DMA,
    ],
)
def _deg_sc(dst3_hbm, out_hbm, didx2, ones_v, zbuf, deg_sh, sem):
    cid = lax.axis_index("c")
    sid = lax.axis_index("s")
    wid = cid * NS + sid
    one16 = jnp.full((LANES,), 1.0, dtype=jnp.float32)

    @pl.loop(0, K)
    def _(r):
        @pl.loop(0, DIM, step=LANES)
        def _(c0):
            ones_v[r, pl.ds(c0, LANES)] = one16

    pltpu.sync_copy(dst3_hbm.at[wid], didx2)
    _zero_fill(zbuf)
    _zero_shared(zbuf, deg_sh, sid)
    plsc.subcore_barrier()

    W = 8

    def _fire(j):
        pltpu.async_copy(ones_v, deg_sh.at[didx2.at[j]], sem, add=True)

    def _drain(j):
        pltpu.make_async_copy(ones_v, deg_sh.at[didx2.at[j]], sem).wait()

    @pl.loop(0, W)
    def _(j):
        _fire(j)

    @pl.loop(0, CPT - W)
    def _(j):
        _drain(j)
        _fire(j + W)

    @pl.loop(CPT - W, CPT)
    def _(j):
        _drain(j)

    plsc.subcore_barrier()
    _copy_out(deg_sh, out_hbm, cid, sid)


@functools.partial(
    pl.kernel,
    out_type=jax.ShapeDtypeStruct((NC, N_NODES, DIM), jnp.float32),
    mesh=_mesh,
    scratch_types=[
        pltpu.VMEM((CPT, K), jnp.int32),
        pltpu.VMEM((1, K), jnp.int32),
        pltpu.VMEM((1, K), jnp.int32),
        pltpu.VMEM((K, DIM), jnp.float32),
        pltpu.VMEM((K, DIM), jnp.float32),
        pltpu.VMEM_SHARED((N_NODES, DIM), jnp.float32),
        pltpu.SemaphoreType.DMA,
        pltpu.SemaphoreType.DMA,
        pltpu.SemaphoreType.DMA,
        pltpu.SemaphoreType.DMA,
    ],
)
def _agg_sc(u_hbm, src3_hbm, dst3_hbm, out_hbm,
            sidx2, didxA, didxB, rows0, rows1, agg_sh,
            sem0, sem1, semA, semB):
    cid = lax.axis_index("c")
    sid = lax.axis_index("s")
    wid = cid * NS + sid

    pltpu.sync_copy(src3_hbm.at[wid], sidx2)
    zero16 = jnp.zeros((LANES,), dtype=jnp.float32)

    @pl.loop(0, K)
    def _(r):
        @pl.loop(0, DIM, step=LANES)
        def _(c0):
            rows0[r, pl.ds(c0, LANES)] = zero16

    @pl.loop(0, RSPAN, step=K)
    def _(r):
        pltpu.sync_copy(rows0, agg_sh.at[pl.ds(sid * RSTEP + r, K)])

    plsc.subcore_barrier()

    def _dload(j, dbuf, dsem):
        pltpu.async_copy(dst3_hbm.at[wid, pl.ds(j, 1)], dbuf, dsem)

    def _dwait(j, dbuf, dsem):
        pltpu.make_async_copy(dst3_hbm.at[wid, pl.ds(j, 1)], dbuf, dsem).wait()

    def _gather(j, buf, sem):
        pltpu.async_copy(u_hbm.at[sidx2.at[j]], buf, sem)

    def _gwait(j, buf, sem):
        pltpu.make_async_copy(u_hbm.at[sidx2.at[j]], buf, sem).wait()

    def _scatter(j, buf, dbuf, dsem):
        _dwait(j, dbuf, dsem)
        pltpu.sync_copy(buf, agg_sh.at[dbuf.at[0]], add=True)

    _dload(0, didxA, semA)
    _gather(0, rows0, sem0)

    @pl.loop(0, CPT - 3, step=2)
    def _(j):
        _dload(j + 1, didxB, semB)
        _gather(j + 1, rows1, sem1)
        _gwait(j, rows0, sem0)
        _scatter(j, rows0, didxA, semA)
        _dload(j + 2, didxA, semA)
        _gather(j + 2, rows0, sem0)
        _gwait(j + 1, rows1, sem1)
        _scatter(j + 1, rows1, didxB, semB)

    _dload(CPT - 2, didxB, semB)
    _gather(CPT - 2, rows1, sem1)
    _gwait(CPT - 3, rows0, sem0)
    _scatter(CPT - 3, rows0, didxA, semA)
    _dload(CPT - 1, didxA, semA)
    _gather(CPT - 1, rows0, sem0)
    _gwait(CPT - 2, rows1, sem1)
    _scatter(CPT - 2, rows1, didxB, semB)
    _gwait(CPT - 1, rows0, sem0)
    _scatter(CPT - 1, rows0, didxA, semA)

    plsc.subcore_barrier()
    _copy_out(agg_sh, out_hbm, cid, sid)


def _dinv_from(degp_ref):
    deg = degp_ref[0] + degp_ref[1] + 1.0
    return lax.rsqrt(deg)


def _tc_in_body(x_ref, win_ref, bin_ref, w1_ref, degp_ref, o_ref):
    dinv = _dinv_from(degp_ref)
    h = (
        jnp.dot(x_ref[...], win_ref[...], preferred_element_type=jnp.float32)
        + bin_ref[...]
    )
    o_ref[...] = (
        jnp.dot(h, w1_ref[...], preferred_element_type=jnp.float32) * dinv
    )


def _tc_layer_body(aggp_ref, u_ref, degp_ref, b_ref, w_ref, o_ref):
    dinv = _dinv_from(degp_ref)
    agg = aggp_ref[0] + aggp_ref[1] + u_ref[...]
    hl = jnp.maximum(agg * dinv + b_ref[...], 0.0)
    o_ref[...] = (
        jnp.dot(hl, w_ref[...], preferred_element_type=jnp.float32) * dinv
    )


def _tc_final_body(aggp_ref, u_ref, degp_ref, b_ref, o_ref):
    dinv = _dinv_from(degp_ref)
    agg = aggp_ref[0] + aggp_ref[1] + u_ref[...]
    o_ref[...] = jnp.maximum(agg * dinv + b_ref[...], 0.0)


_out_nd = jax.ShapeDtypeStruct((N_NODES, DIM), jnp.float32)

_tc_in = pl.pallas_call(_tc_in_body, out_shape=_out_nd)
_tc_layer = pl.pallas_call(_tc_layer_body, out_shape=_out_nd)
_tc_final = pl.pallas_call(_tc_final_body, out_shape=_out_nd)


def kernel(x, edge_index, W_in, b_in, W1, b1, W2, b2):
    src3 = edge_index[0].reshape(NTILE, CPT, K)
    dst3 = edge_index[1].reshape(NTILE, CPT, K)
    b_in2 = b_in.reshape(1, DIM)
    b1_2 = b1.reshape(1, DIM)
    b2_2 = b2.reshape(1, DIM)

    degp = _deg_sc(dst3)
    degs = degp[:, :, 0:1]
    u1 = _tc_in(x, W_in, b_in2, W1, degs)
    aggp1 = _agg_sc(u1, src3, dst3)
    u2 = _tc_layer(aggp1, u1, degs, b1_2, W2)
    aggp2 = _agg_sc(u2, src3, dst3)
    return _tc_final(aggp2, u2, degs, b2_2)


# speedup vs baseline: 28.6158x; 1.0210x over previous
import functools

import jax
import jax.numpy as jnp
from jax import lax
from jax.experimental import pallas as pl
from jax.experimental.pallas import tpu as pltpu
from jax.experimental.pallas import tpu_sc as plsc

N_NODES = 10000
DIM = 128
NUM_EDGES = 320000
NC = 2
NS = 16
LANES = 16
NTILE = NC * NS

EPT = NUM_EDGES // NTILE
K = 80
CPT = EPT // K

RSTEP = 624
RSPAN = 640

_mesh = plsc.VectorSubcoreMesh(core_axis_name="c", subcore_axis_name="s")


def _zero_fill(zbuf):
    zero16 = jnp.zeros((LANES,), dtype=jnp.float32)

    @pl.loop(0, 64)
    def _(r):
        @pl.loop(0, DIM, step=LANES)
        def _(c0):
            zbuf[r, pl.ds(c0, LANES)] = zero16


def _zero_shared(zbuf, sh, sid):
    @pl.loop(0, RSPAN, step=64)
    def _(r):
        pltpu.sync_copy(zbuf, sh.at[pl.ds(sid * RSTEP + r, 64)])


def _copy_out(sh, out_hbm, cid, sid):
    pltpu.sync_copy(
        sh.at[pl.ds(sid * RSTEP, RSPAN)],
        out_hbm.at[cid, pl.ds(sid * RSTEP, RSPAN)],
    )


@functools.partial(
    pl.kernel,
    out_type=jax.ShapeDtypeStruct((NC, N_NODES, DIM), jnp.float32),
    mesh=_mesh,
    scratch_types=[
        pltpu.VMEM((CPT, K), jnp.int32),
        pltpu.VMEM((K, DIM), jnp.float32),
        pltpu.VMEM((64, DIM), jnp.float32),
        pltpu.VMEM_SHARED((N_NODES, DIM), jnp.float32),
        pltpu.SemaphoreType.DMA,
    ],
)
def _deg_sc(dst3_hbm, out_hbm, didx2, ones_v, zbuf, deg_sh, sem):
    cid = lax.axis_index("c")
    sid = lax.axis_index("s")
    wid = cid * NS + sid
    one16 = jnp.full((LANES,), 1.0, dtype=jnp.float32)

    @pl.loop(0, K)
    def _(r):
        @pl.loop(0, DIM, step=LANES)
        def _(c0):
            ones_v[r, pl.ds(c0, LANES)] = one16

    pltpu.sync_copy(dst3_hbm.at[wid], didx2)
    _zero_fill(zbuf)
    _zero_shared(zbuf, deg_sh, sid)
    plsc.subcore_barrier()

    W = 8

    def _fire(j):
        pltpu.async_copy(ones_v, deg_sh.at[didx2.at[j]], sem, add=True)

    def _drain(j):
        pltpu.make_async_copy(ones_v, deg_sh.at[didx2.at[j]], sem).wait()

    @pl.loop(0, W)
    def _(j):
        _fire(j)

    @pl.loop(0, CPT - W)
    def _(j):
        _drain(j)
        _fire(j + W)

    @pl.loop(CPT - W, CPT)
    def _(j):
        _drain(j)

    plsc.subcore_barrier()
    _copy_out(deg_sh, out_hbm, cid, sid)


@functools.partial(
    pl.kernel,
    out_type=jax.ShapeDtypeStruct((NC, N_NODES, DIM), jnp.float32),
    mesh=_mesh,
    scratch_types=[
        pltpu.VMEM((CPT, K), jnp.int32),
        pltpu.VMEM((1, K), jnp.int32),
        pltpu.VMEM((1, K), jnp.int32),
        pltpu.VMEM((K, DIM), jnp.float32),
        pltpu.VMEM((K, DIM), jnp.float32),
        pltpu.VMEM_SHARED((N_NODES, DIM), jnp.float32),
        pltpu.SemaphoreType.DMA,
        pltpu.SemaphoreType.DMA,
        pltpu.SemaphoreType.DMA,
        pltpu.SemaphoreType.DMA,
        pltpu.SemaphoreType.DMA,
        pltpu.SemaphoreType.DMA,
    ],
)
def _agg_sc(u_hbm, src3_hbm, dst3_hbm, out_hbm,
            sidx2, didxA, didxB, rows0, rows1, agg_sh,
            sem0, sem1, semA, semB, sem0b, sem1b):
    cid = lax.axis_index("c")
    sid = lax.axis_index("s")
    wid = cid * NS + sid

    pltpu.sync_copy(src3_hbm.at[wid], sidx2)
    zero16 = jnp.zeros((LANES,), dtype=jnp.float32)

    @pl.loop(0, K)
    def _(r):
        @pl.loop(0, DIM, step=LANES)
        def _(c0):
            rows0[r, pl.ds(c0, LANES)] = zero16

    @pl.loop(0, RSPAN, step=K)
    def _(r):
        pltpu.sync_copy(rows0, agg_sh.at[pl.ds(sid * RSTEP + r, K)])

    plsc.subcore_barrier()

    def _dload(j, dbuf, dsem):
        pltpu.async_copy(dst3_hbm.at[wid, pl.ds(j, 1)], dbuf, dsem)

    def _dwait(j, dbuf, dsem):
        pltpu.make_async_copy(dst3_hbm.at[wid, pl.ds(j, 1)], dbuf, dsem).wait()

    H = K // 2

    def _gather(j, buf, sems):
        pltpu.async_copy(
            u_hbm.at[sidx2.at[j, pl.ds(0, H)]], buf.at[pl.ds(0, H)], sems[0])
        pltpu.async_copy(
            u_hbm.at[sidx2.at[j, pl.ds(H, H)]], buf.at[pl.ds(H, H)], sems[1])

    def _gwait(j, buf, sems):
        pltpu.make_async_copy(
            u_hbm.at[sidx2.at[j, pl.ds(0, H)]], buf.at[pl.ds(0, H)],
            sems[0]).wait()
        pltpu.make_async_copy(
            u_hbm.at[sidx2.at[j, pl.ds(H, H)]], buf.at[pl.ds(H, H)],
            sems[1]).wait()

    def _scatter(j, buf, dbuf, dsem):
        _dwait(j, dbuf, dsem)
        pltpu.sync_copy(buf, agg_sh.at[dbuf.at[0]], add=True)

    _dload(0, didxA, semA)
    _gather(0, rows0, (sem0, sem0b))

    @pl.loop(0, CPT - 3, step=2)
    def _(j):
        _dload(j + 1, didxB, semB)
        _gather(j + 1, rows1, (sem1, sem1b))
        _gwait(j, rows0, (sem0, sem0b))
        _scatter(j, rows0, didxA, semA)
        _dload(j + 2, didxA, semA)
        _gather(j + 2, rows0, (sem0, sem0b))
        _gwait(j + 1, rows1, (sem1, sem1b))
        _scatter(j + 1, rows1, didxB, semB)

    _dload(CPT - 2, didxB, semB)
    _gather(CPT - 2, rows1, (sem1, sem1b))
    _gwait(CPT - 3, rows0, (sem0, sem0b))
    _scatter(CPT - 3, rows0, didxA, semA)
    _dload(CPT - 1, didxA, semA)
    _gather(CPT - 1, rows0, (sem0, sem0b))
    _gwait(CPT - 2, rows1, (sem1, sem1b))
    _scatter(CPT - 2, rows1, didxB, semB)
    _gwait(CPT - 1, rows0, (sem0, sem0b))
    _scatter(CPT - 1, rows0, didxA, semA)

    plsc.subcore_barrier()
    _copy_out(agg_sh, out_hbm, cid, sid)


def _dinv_from(degp_ref):
    deg = degp_ref[0] + degp_ref[1] + 1.0
    return lax.rsqrt(deg)


def _tc_in_body(x_ref, win_ref, bin_ref, w1_ref, degp_ref, o_ref):
    dinv = _dinv_from(degp_ref)
    h = (
        jnp.dot(x_ref[...], win_ref[...], preferred_element_type=jnp.float32)
        + bin_ref[...]
    )
    o_ref[...] = (
        jnp.dot(h, w1_ref[...], preferred_element_type=jnp.float32) * dinv
    )


def _tc_layer_body(aggp_ref, u_ref, degp_ref, b_ref, w_ref, o_ref):
    dinv = _dinv_from(degp_ref)
    agg = aggp_ref[0] + aggp_ref[1] + u_ref[...]
    hl = jnp.maximum(agg * dinv + b_ref[...], 0.0)
    o_ref[...] = (
        jnp.dot(hl, w_ref[...], preferred_element_type=jnp.float32) * dinv
    )


def _tc_final_body(aggp_ref, u_ref, degp_ref, b_ref, o_ref):
    dinv = _dinv_from(degp_ref)
    agg = aggp_ref[0] + aggp_ref[1] + u_ref[...]
    o_ref[...] = jnp.maximum(agg * dinv + b_ref[...], 0.0)


_out_nd = jax.ShapeDtypeStruct((N_NODES, DIM), jnp.float32)

_tc_in = pl.pallas_call(_tc_in_body, out_shape=_out_nd)
_tc_layer = pl.pallas_call(_tc_layer_body, out_shape=_out_nd)
_tc_final = pl.pallas_call(_tc_final_body, out_shape=_out_nd)


def kernel(x, edge_index, W_in, b_in, W1, b1, W2, b2):
    src3 = edge_index[0].reshape(NTILE, CPT, K)
    dst3 = edge_index[1].reshape(NTILE, CPT, K)
    b_in2 = b_in.reshape(1, DIM)
    b1_2 = b1.reshape(1, DIM)
    b2_2 = b2.reshape(1, DIM)

    degp = _deg_sc(dst3)
    degs = degp[:, :, 0:1]
    u1 = _tc_in(x, W_in, b_in2, W1, degs)
    aggp1 = _agg_sc(u1, src3, dst3)
    u2 = _tc_layer(aggp1, u1, degs, b1_2, W2)
    aggp2 = _agg_sc(u2, src3, dst3)
    return _tc_final(aggp2, u2, degs, b2_2)
